# Initial kernel scaffold; baseline (speedup 1.0000x reference)
import jax
import jax.numpy as jnp
from jax.experimental import pallas as pl


def kernel(input_ids, attention_mask, embed_table, W1, b1, W2, b2):
    raise NotImplementedError("write your pallas kernel here")


# speedup vs baseline: 3.6819x; 3.6819x over previous
import functools

import jax
import jax.numpy as jnp
from jax import lax
from jax.experimental import pallas as pl
from jax.experimental.pallas import tpu as pltpu
from jax.experimental.pallas import tpu_sc as plsc

VOCAB = 100000
HIDDEN = 64
ROWS_BLOCK = 10000

NUM_CORES = 2
NUM_SUBCORES = 16
NUM_WORKERS = NUM_CORES * NUM_SUBCORES

TOTAL_IDS = 4096 * 50
CHUNK = 128
N_CHUNKS = TOTAL_IDS // CHUNK
CHUNKS_PER_W = N_CHUNKS // NUM_WORKERS
PHASE = 10
N_PHASES = CHUNKS_PER_W // PHASE
ROWS_PER_PHASE = PHASE * CHUNK


def _mlp_body(e_ref, w1_ref, b1_ref, w2_ref, b2_ref, o_ref):
    e = e_ref[...]
    h = lax.dot_general(e, w1_ref[...], (((1,), (1,)), ((), ())),
                        preferred_element_type=jnp.float32)
    h = jnp.tanh(h + b1_ref[...])
    o = lax.dot_general(h, w2_ref[...], (((1,), (1,)), ((), ())),
                        preferred_element_type=jnp.float32)
    o_ref[...] = o + b2_ref[...]


def _transform_table(embed_table, w1, b1, w2, b2):
    grid = VOCAB // ROWS_BLOCK
    return pl.pallas_call(
        _mlp_body,
        grid=(grid,),
        in_specs=[
            pl.BlockSpec((ROWS_BLOCK, HIDDEN), lambda i: (i, 0)),
            pl.BlockSpec((HIDDEN, HIDDEN), lambda i: (0, 0)),
            pl.BlockSpec((1, HIDDEN), lambda i: (0, 0)),
            pl.BlockSpec((HIDDEN, HIDDEN), lambda i: (0, 0)),
            pl.BlockSpec((1, HIDDEN), lambda i: (0, 0)),
        ],
        out_specs=pl.BlockSpec((ROWS_BLOCK, HIDDEN), lambda i: (i, 0)),
        out_shape=jax.ShapeDtypeStruct((VOCAB, HIDDEN), jnp.float32),
    )(embed_table, w1, b1.reshape(1, HIDDEN), w2, b2.reshape(1, HIDDEN))


IDS_PER_W = CHUNKS_PER_W * CHUNK


def _gather_body(ids_hbm, t_hbm, out_hbm, idx_v, rows_v, gsem):
    wid = lax.axis_index("s") * NUM_CORES + lax.axis_index("c")
    pltpu.sync_copy(ids_hbm.at[pl.ds(wid * IDS_PER_W, IDS_PER_W)], idx_v)

    def phase(p, carry):
        descs = []
        for b in range(PHASE):
            d = pltpu.async_copy(
                t_hbm.at[idx_v.at[pl.ds((p * PHASE + b) * CHUNK, CHUNK)]],
                rows_v.at[pl.ds(b * CHUNK, CHUNK)],
                gsem,
            )
            descs.append(d)
        for d in descs:
            d.wait()
        out_row = wid * IDS_PER_W + p * ROWS_PER_PHASE
        pltpu.sync_copy(rows_v, out_hbm.at[pl.ds(out_row, ROWS_PER_PHASE)])
        return carry

    lax.fori_loop(0, N_PHASES, phase, 0)


def _gather_rows(ids_flat, table_t):
    mesh = plsc.VectorSubcoreMesh(
        core_axis_name="c", subcore_axis_name="s",
        num_cores=NUM_CORES, num_subcores=NUM_SUBCORES,
    )
    k = pl.kernel(
        _gather_body,
        out_type=jax.ShapeDtypeStruct((TOTAL_IDS, HIDDEN), jnp.float32),
        mesh=mesh,
        scratch_types=[
            pltpu.VMEM((IDS_PER_W,), jnp.int32),
            pltpu.VMEM((ROWS_PER_PHASE, HIDDEN), jnp.float32),
            pltpu.SemaphoreType.DMA,
        ],
        compiler_params=pltpu.CompilerParams(use_tc_tiling_on_sc=False),
    )
    return k(ids_flat, table_t)


def kernel(input_ids, attention_mask, embed_table, W1, b1, W2, b2):
    del attention_mask
    B, L = input_ids.shape
    table_t = _transform_table(embed_table, W1, b1, W2, b2)
    ids_flat = input_ids.astype(jnp.int32).reshape(TOTAL_IDS)
    out2d = _gather_rows(ids_flat, table_t)
    return out2d.reshape(B, L, HIDDEN)


# speedup vs baseline: 5.2209x; 1.4180x over previous
import jax
import jax.numpy as jnp
from jax import lax
from jax.experimental import pallas as pl
from jax.experimental.pallas import tpu as pltpu
from jax.experimental.pallas import tpu_sc as plsc

VOCAB = 100000
HIDDEN = 64
B = 4096
L = 50

BLK_V = 12800
HBLK = BLK_V // 2
VPAD = 102400


def _mlp_body(e_ref, w1_ref, b1_ref, w2_ref, b2_ref, o_ref):
    e = e_ref[...]
    h = lax.dot_general(e, w1_ref[...], (((0,), (1,)), ((), ())),
                        preferred_element_type=jnp.float32)
    h = jnp.tanh(h + b1_ref[...])
    z = lax.dot_general(h, w2_ref[...], (((1,), (1,)), ((), ())),
                        preferred_element_type=jnp.float32)
    z = z + b2_ref[...]
    o_ref[...] = jnp.concatenate([z[:HBLK], z[HBLK:]], axis=1)


def _transform_table(embed_table_t, w1, b1, w2, b2):
    grid = VPAD // BLK_V
    return pl.pallas_call(
        _mlp_body,
        grid=(grid,),
        in_specs=[
            pl.BlockSpec((HIDDEN, BLK_V), lambda i: (0, i)),
            pl.BlockSpec((HIDDEN, HIDDEN), lambda i: (0, 0)),
            pl.BlockSpec((1, HIDDEN), lambda i: (0, 0)),
            pl.BlockSpec((HIDDEN, HIDDEN), lambda i: (0, 0)),
            pl.BlockSpec((1, HIDDEN), lambda i: (0, 0)),
        ],
        out_specs=pl.BlockSpec((HBLK, 2 * HIDDEN), lambda i: (i, 0)),
        out_shape=jax.ShapeDtypeStruct((VPAD // 2, 2 * HIDDEN), jnp.float32),
    )(embed_table_t, w1, b1.reshape(1, HIDDEN), w2, b2.reshape(1, HIDDEN))


def _sigma(w):
    r = w % BLK_V
    return (w - r) + 2 * (r % HBLK) + r // HBLK


NUM_CORES = 2
NUM_SUBCORES = 16
NUM_WORKERS = NUM_CORES * NUM_SUBCORES

TOTAL_IDS = B * L
CHUNK = 128
IDS_PER_W = TOTAL_IDS // NUM_WORKERS
CHUNKS_PER_W = IDS_PER_W // CHUNK
PHASE = 10
N_PHASES = CHUNKS_PER_W // PHASE
ROWS_PER_PHASE = PHASE * CHUNK


def _gather_body(ids_hbm, t_hbm, out_hbm, idx_v, rows_v, gsem):
    wid = lax.axis_index("s") * NUM_CORES + lax.axis_index("c")
    pltpu.sync_copy(ids_hbm.at[pl.ds(wid * IDS_PER_W, IDS_PER_W)], idx_v)

    def phase(p, carry):
        descs = []
        for bb in range(PHASE):
            d = pltpu.async_copy(
                t_hbm.at[idx_v.at[pl.ds((p * PHASE + bb) * CHUNK, CHUNK)]],
                rows_v.at[pl.ds(bb * CHUNK, CHUNK)],
                gsem,
            )
            descs.append(d)
        for d in descs:
            d.wait()
        out_row = wid * IDS_PER_W + p * ROWS_PER_PHASE
        pltpu.sync_copy(rows_v, out_hbm.at[pl.ds(out_row, ROWS_PER_PHASE)])
        return carry

    lax.fori_loop(0, N_PHASES, phase, 0)


def _gather_rows(ids_flat, t_lin):
    mesh = plsc.VectorSubcoreMesh(
        core_axis_name="c", subcore_axis_name="s",
        num_cores=NUM_CORES, num_subcores=NUM_SUBCORES,
    )
    k = pl.kernel(
        _gather_body,
        out_type=jax.ShapeDtypeStruct((TOTAL_IDS, HIDDEN), jnp.float32),
        mesh=mesh,
        scratch_types=[
            pltpu.VMEM((IDS_PER_W,), jnp.int32),
            pltpu.VMEM((ROWS_PER_PHASE, HIDDEN), jnp.float32),
            pltpu.SemaphoreType.DMA,
        ],
        compiler_params=pltpu.CompilerParams(use_tc_tiling_on_sc=False),
    )
    return k(ids_flat, t_lin)


HB = B // 2


def _delin_body(g_ref, o_ref):
    g = g_ref[...]
    ze = g[:, :HIDDEN].T
    zo = g[:, HIDDEN:].T
    o_ref[...] = jnp.concatenate([ze, zo], axis=1).reshape(1, HIDDEN, B)


def _delinearize(g2):
    return pl.pallas_call(
        _delin_body,
        grid=(L,),
        in_specs=[pl.BlockSpec((HB, 2 * HIDDEN), lambda i: (i, 0))],
        out_specs=pl.BlockSpec((1, HIDDEN, B), lambda i: (i, 0, 0)),
        out_shape=jax.ShapeDtypeStruct((L, HIDDEN, B), jnp.float32),
    )(g2)


def kernel(input_ids, attention_mask, embed_table, W1, b1, W2, b2):
    del attention_mask
    table_t = _transform_table(embed_table.T, W1, b1, W2, b2)
    t_lin = table_t.reshape(VPAD, HIDDEN)

    ids_q = (input_ids.astype(jnp.int32).T
             .reshape(L, 2, HB).transpose(0, 2, 1).reshape(L, B))
    ids_flat = _sigma(ids_q).reshape(TOTAL_IDS)

    g = _gather_rows(ids_flat, t_lin)
    g2 = g.reshape(TOTAL_IDS // 2, 2 * HIDDEN)
    p = _delinearize(g2)
    return jnp.transpose(p, (2, 0, 1))


# speedup vs baseline: 7.0987x; 1.3597x over previous
import jax
import jax.numpy as jnp
from jax import lax
from jax.experimental import pallas as pl
from jax.experimental.pallas import tpu as pltpu
from jax.experimental.pallas import tpu_sc as plsc

VOCAB = 100000
HIDDEN = 64
B = 4096
L = 50
HB = B // 2

BLK_V = 16384
HBLK = BLK_V // 2
VPAD = 114688


def _mlp_body(e_ref, w1_ref, b1_ref, w2_ref, b2_ref, o_ref):
    e = e_ref[...]
    h = lax.dot_general(e, w1_ref[...], (((0,), (1,)), ((), ())),
                        preferred_element_type=jnp.float32)
    h = jnp.tanh(h + b1_ref[...])
    z = lax.dot_general(h, w2_ref[...], (((1,), (1,)), ((), ())),
                        preferred_element_type=jnp.float32)
    z = z + b2_ref[...]
    o_ref[...] = jnp.concatenate([z[:HBLK], z[HBLK:]], axis=1)


def _transform_table(embed_table_t, w1, b1, w2, b2):
    grid = VPAD // BLK_V
    return pl.pallas_call(
        _mlp_body,
        grid=(grid,),
        in_specs=[
            pl.BlockSpec((HIDDEN, BLK_V), lambda i: (0, i)),
            pl.BlockSpec((HIDDEN, HIDDEN), lambda i: (0, 0)),
            pl.BlockSpec((1, HIDDEN), lambda i: (0, 0)),
            pl.BlockSpec((HIDDEN, HIDDEN), lambda i: (0, 0)),
            pl.BlockSpec((1, HIDDEN), lambda i: (0, 0)),
        ],
        out_specs=pl.BlockSpec((HBLK, 2 * HIDDEN), lambda i: (i, 0)),
        out_shape=jax.ShapeDtypeStruct((VPAD // 2, 2 * HIDDEN), jnp.float32),
    )(embed_table_t, w1, b1.reshape(1, HIDDEN), w2, b2.reshape(1, HIDDEN))


NUM_CORES = 2
NUM_SUBCORES = 16
NUM_WORKERS = NUM_CORES * NUM_SUBCORES

TOTAL_IDS = B * L
CHUNK = 128
IDS_PER_W = TOTAL_IDS // NUM_WORKERS
CHUNKS_PER_W = IDS_PER_W // CHUNK
PHASE = 10
N_PHASES = CHUNKS_PER_W // PHASE
ROWS_PER_PHASE = PHASE * CHUNK
RAW_N = 3 * B
N_GROUPS = IDS_PER_W // 16


def _gather_body(ids_hbm, t_hbm, out_hbm, raw_v, idx_v, rows_v, gsem):
    wid = lax.axis_index("s") * NUM_CORES + lax.axis_index("c")
    p0 = wid * IDS_PER_W
    lbase = jnp.minimum(p0 // B, L - 3)
    pltpu.sync_copy(ids_hbm.at[pl.ds(lbase * B, RAW_N)], raw_v)

    i16 = lax.iota(jnp.int32, 16)
    pattern = HB * (i16 & 1) + (i16 >> 1)

    def build(g, carry):
        qg = p0 + 16 * g
        lrel = (qg >> 12) - lbase
        qr = qg & (B - 1)
        c0 = lrel * B + (qr >> 1)
        w = plsc.load_gather(raw_v, [pattern + c0])
        r = w & (BLK_V - 1)
        sig = (w - r) + 2 * (r & (HBLK - 1)) + (r >> 13)
        idx_v[pl.ds(16 * g, 16)] = sig
        return carry

    lax.fori_loop(0, N_GROUPS, build, 0)

    def phase(p, carry):
        descs = []
        for bb in range(PHASE):
            d = pltpu.async_copy(
                t_hbm.at[idx_v.at[pl.ds((p * PHASE + bb) * CHUNK, CHUNK)]],
                rows_v.at[pl.ds(bb * CHUNK, CHUNK)],
                gsem,
            )
            descs.append(d)
        for d in descs:
            d.wait()
        out_row = p0 + p * ROWS_PER_PHASE
        pltpu.sync_copy(rows_v, out_hbm.at[pl.ds(out_row, ROWS_PER_PHASE)])
        return carry

    lax.fori_loop(0, N_PHASES, phase, 0)


def _gather_rows(ids_flat, t_lin):
    mesh = plsc.VectorSubcoreMesh(
        core_axis_name="c", subcore_axis_name="s",
        num_cores=NUM_CORES, num_subcores=NUM_SUBCORES,
    )
    k = pl.kernel(
        _gather_body,
        out_type=jax.ShapeDtypeStruct((TOTAL_IDS, HIDDEN), jnp.float32),
        mesh=mesh,
        scratch_types=[
            pltpu.VMEM((RAW_N,), jnp.int32),
            pltpu.VMEM((IDS_PER_W,), jnp.int32),
            pltpu.VMEM((ROWS_PER_PHASE, HIDDEN), jnp.float32),
            pltpu.SemaphoreType.DMA,
        ],
        compiler_params=pltpu.CompilerParams(use_tc_tiling_on_sc=False,
                                             needs_layout_passes=False),
    )
    return k(ids_flat, t_lin)


def _delin_body(g_ref, o_ref):
    g = g_ref[...]
    ze = g[:, :HIDDEN].T
    zo = g[:, HIDDEN:].T
    o_ref[...] = jnp.concatenate([ze, zo], axis=1).reshape(1, HIDDEN, B)


def _delinearize(g2):
    return pl.pallas_call(
        _delin_body,
        grid=(L,),
        in_specs=[pl.BlockSpec((HB, 2 * HIDDEN), lambda i: (i, 0))],
        out_specs=pl.BlockSpec((1, HIDDEN, B), lambda i: (i, 0, 0)),
        out_shape=jax.ShapeDtypeStruct((L, HIDDEN, B), jnp.float32),
    )(g2)


def kernel(input_ids, attention_mask, embed_table, W1, b1, W2, b2):
    del attention_mask
    table_t = _transform_table(embed_table.T, W1, b1, W2, b2)
    t_lin = table_t.reshape(VPAD, HIDDEN)

    ids_flat = input_ids.astype(jnp.int32).T.reshape(TOTAL_IDS)

    g = _gather_rows(ids_flat, t_lin)
    g2 = g.reshape(TOTAL_IDS // 2, 2 * HIDDEN)
    p = _delinearize(g2)
    return jnp.transpose(p, (2, 0, 1))


# speedup vs baseline: 7.4424x; 1.0484x over previous
import jax
import jax.numpy as jnp
from jax import lax
from jax.experimental import pallas as pl
from jax.experimental.pallas import tpu as pltpu
from jax.experimental.pallas import tpu_sc as plsc

VOCAB = 100000
HIDDEN = 64
B = 4096
L = 50
HB = B // 2

BLK_V = 16384
HBLK = BLK_V // 2
VPAD = 114688


def _mlp_body(e_ref, w1_ref, b1_ref, w2_ref, b2_ref, o_ref):
    e = e_ref[...]
    h = lax.dot_general(e, w1_ref[...], (((0,), (1,)), ((), ())),
                        preferred_element_type=jnp.float32)
    h = jnp.tanh(h + b1_ref[...])
    z = lax.dot_general(h, w2_ref[...], (((1,), (1,)), ((), ())),
                        preferred_element_type=jnp.float32)
    z = z + b2_ref[...]
    o_ref[...] = jnp.concatenate([z[:HBLK], z[HBLK:]], axis=1)


def _transform_table(embed_table_t, w1, b1, w2, b2):
    grid = VPAD // BLK_V
    return pl.pallas_call(
        _mlp_body,
        grid=(grid,),
        in_specs=[
            pl.BlockSpec((HIDDEN, BLK_V), lambda i: (0, i)),
            pl.BlockSpec((HIDDEN, HIDDEN), lambda i: (0, 0)),
            pl.BlockSpec((1, HIDDEN), lambda i: (0, 0)),
            pl.BlockSpec((HIDDEN, HIDDEN), lambda i: (0, 0)),
            pl.BlockSpec((1, HIDDEN), lambda i: (0, 0)),
        ],
        out_specs=pl.BlockSpec((HBLK, 2 * HIDDEN), lambda i: (i, 0)),
        out_shape=jax.ShapeDtypeStruct((VPAD // 2, 2 * HIDDEN), jnp.float32),
    )(embed_table_t, w1, b1.reshape(1, HIDDEN), w2, b2.reshape(1, HIDDEN))


NUM_CORES = 2
NUM_SUBCORES = 16
NUM_WORKERS = NUM_CORES * NUM_SUBCORES

TOTAL_IDS = B * L
CHUNK = 128
IDS_PER_W = TOTAL_IDS // NUM_WORKERS
CHUNKS_PER_W = IDS_PER_W // CHUNK
PHASE = 10
N_PHASES = CHUNKS_PER_W // PHASE
ROWS_PER_PHASE = PHASE * CHUNK
RAW_N = 3 * B
N_GROUPS = IDS_PER_W // 16


def _gather_body(ids_hbm, t_hbm, out_hbm, raw_v, idx_v, rows_v, gsem):
    wid = lax.axis_index("s") * NUM_CORES + lax.axis_index("c")
    p0 = wid * IDS_PER_W
    lbase = jnp.minimum(p0 // B, L - 3)
    pltpu.sync_copy(ids_hbm.at[pl.ds(lbase * B, RAW_N)], raw_v)

    i16 = lax.iota(jnp.int32, 16)
    pattern = HB * (i16 & 1) + (i16 >> 1)

    def build(g, carry):
        qg = p0 + 16 * g
        lrel = (qg >> 12) - lbase
        qr = qg & (B - 1)
        c0 = lrel * B + (qr >> 1)
        w = plsc.load_gather(raw_v, [pattern + c0])
        r = w & (BLK_V - 1)
        sig = (w - r) + 2 * (r & (HBLK - 1)) + (r >> 13)
        idx_v[pl.ds(16 * g, 16)] = sig
        return carry

    lax.fori_loop(0, N_GROUPS, build, 0)

    def phase(p, carry):
        descs = []
        for bb in range(PHASE):
            d = pltpu.async_copy(
                t_hbm.at[idx_v.at[pl.ds((p * PHASE + bb) * CHUNK, CHUNK)]],
                rows_v.at[pl.ds(bb * CHUNK, CHUNK)],
                gsem,
            )
            descs.append(d)
        for d in descs:
            d.wait()
        out_row = p0 + p * ROWS_PER_PHASE
        pltpu.sync_copy(rows_v, out_hbm.at[pl.ds(out_row, ROWS_PER_PHASE)])
        return carry

    lax.fori_loop(0, N_PHASES, phase, 0)


def _gather_rows(ids_flat, t_lin):
    mesh = plsc.VectorSubcoreMesh(
        core_axis_name="c", subcore_axis_name="s",
        num_cores=NUM_CORES, num_subcores=NUM_SUBCORES,
    )
    k = pl.kernel(
        _gather_body,
        out_type=jax.ShapeDtypeStruct((TOTAL_IDS, HIDDEN), jnp.float32),
        mesh=mesh,
        scratch_types=[
            pltpu.VMEM((RAW_N,), jnp.int32),
            pltpu.VMEM((IDS_PER_W,), jnp.int32),
            pltpu.VMEM((ROWS_PER_PHASE, HIDDEN), jnp.float32),
            pltpu.SemaphoreType.DMA,
        ],
        compiler_params=pltpu.CompilerParams(use_tc_tiling_on_sc=False,
                                             needs_layout_passes=False),
    )
    return k(ids_flat, t_lin)


def _delin_body(g_ref, o_ref):
    g = g_ref[...]
    ii = lax.broadcasted_iota(jnp.int32, (HIDDEN, HIDDEN), 0)
    jj = lax.broadcasted_iota(jnp.int32, (HIDDEN, HIDDEN), 1)
    eye = jnp.where(ii == jj, 1.0, 0.0).astype(jnp.float32)
    ze = lax.dot_general(eye, g[:, :HIDDEN], (((1,), (1,)), ((), ())),
                         preferred_element_type=jnp.float32)
    zo = lax.dot_general(eye, g[:, HIDDEN:], (((1,), (1,)), ((), ())),
                         preferred_element_type=jnp.float32)
    o_ref[...] = jnp.concatenate([ze, zo], axis=1).reshape(1, HIDDEN, B)


def _delinearize(g2):
    return pl.pallas_call(
        _delin_body,
        grid=(L,),
        in_specs=[pl.BlockSpec((HB, 2 * HIDDEN), lambda i: (i, 0))],
        out_specs=pl.BlockSpec((1, HIDDEN, B), lambda i: (i, 0, 0)),
        out_shape=jax.ShapeDtypeStruct((L, HIDDEN, B), jnp.float32),
    )(g2)


def kernel(input_ids, attention_mask, embed_table, W1, b1, W2, b2):
    del attention_mask
    table_t = _transform_table(embed_table.T, W1, b1, W2, b2)
    t_lin = table_t.reshape(VPAD, HIDDEN)

    ids_flat = input_ids.astype(jnp.int32).T.reshape(TOTAL_IDS)

    g = _gather_rows(ids_flat, t_lin)
    g2 = g.reshape(TOTAL_IDS // 2, 2 * HIDDEN)
    p = _delinearize(g2)
    return jnp.transpose(p, (2, 0, 1))


# speedup vs baseline: 8.2906x; 1.1140x over previous
import jax
import jax.numpy as jnp
from jax import lax
from jax.experimental import pallas as pl
from jax.experimental.pallas import tpu as pltpu
from jax.experimental.pallas import tpu_sc as plsc

VOCAB = 100000
HIDDEN = 64
B = 4096
L = 50
HB = B // 2

BLK_V = 16384
HBLK = BLK_V // 2
VPAD = 114688


def _mlp_body(e_ref, w1_ref, b1_ref, w2_ref, b2_ref, o_ref):
    e = e_ref[...]
    h = lax.dot_general(e, w1_ref[...], (((0,), (1,)), ((), ())),
                        preferred_element_type=jnp.float32)
    h = jnp.tanh(h + b1_ref[...])
    z = lax.dot_general(h, w2_ref[...], (((1,), (1,)), ((), ())),
                        preferred_element_type=jnp.float32)
    z = z + b2_ref[...]
    o_ref[...] = jnp.concatenate([z[:HBLK], z[HBLK:]], axis=1)


def _transform_table(embed_table_t, w1, b1, w2, b2):
    grid = VPAD // BLK_V
    return pl.pallas_call(
        _mlp_body,
        grid=(grid,),
        in_specs=[
            pl.BlockSpec((HIDDEN, BLK_V), lambda i: (0, i)),
            pl.BlockSpec((HIDDEN, HIDDEN), lambda i: (0, 0)),
            pl.BlockSpec((1, HIDDEN), lambda i: (0, 0)),
            pl.BlockSpec((HIDDEN, HIDDEN), lambda i: (0, 0)),
            pl.BlockSpec((1, HIDDEN), lambda i: (0, 0)),
        ],
        out_specs=pl.BlockSpec((HBLK, 2 * HIDDEN), lambda i: (i, 0)),
        out_shape=jax.ShapeDtypeStruct((VPAD // 2, 2 * HIDDEN), jnp.float32),
    )(embed_table_t, w1, b1.reshape(1, HIDDEN), w2, b2.reshape(1, HIDDEN))


NUM_CORES = 2
NUM_SUBCORES = 16
NUM_WORKERS = NUM_CORES * NUM_SUBCORES

TOTAL_IDS = B * L
CHUNK = 128
IDS_PER_W = TOTAL_IDS // NUM_WORKERS
CHUNKS_PER_W = IDS_PER_W // CHUNK
PHASE = 10
N_PHASES = CHUNKS_PER_W // PHASE
ROWS_PER_PHASE = PHASE * CHUNK
RAW_N = 3 * B
N_GROUPS = IDS_PER_W // 16


def _gather_body(ids_hbm, t_hbm, out_hbm, raw_v, idx_v, rows_v, gsem):
    wid = lax.axis_index("s") * NUM_CORES + lax.axis_index("c")
    p0 = wid * IDS_PER_W
    lbase = jnp.minimum(p0 // B, L - 3)
    pltpu.sync_copy(ids_hbm.at[pl.ds(lbase * B, RAW_N)], raw_v)

    i16 = lax.iota(jnp.int32, 16)
    pattern = HB * (i16 & 1) + (i16 >> 1)

    def build(g, carry):
        qg = p0 + 16 * g
        lrel = (qg >> 12) - lbase
        qr = qg & (B - 1)
        c0 = lrel * B + (qr >> 1)
        w = plsc.load_gather(raw_v, [pattern + c0])
        r = w & (BLK_V - 1)
        sig = (w - r) + 2 * (r & (HBLK - 1)) + (r >> 13)
        idx_v[pl.ds(16 * g, 16)] = sig
        return carry

    lax.fori_loop(0, N_GROUPS, build, 0)

    def phase(p, carry):
        descs = []
        for bb in range(PHASE):
            d = pltpu.async_copy(
                t_hbm.at[idx_v.at[pl.ds((p * PHASE + bb) * CHUNK, CHUNK)]],
                rows_v.at[pl.ds(bb * CHUNK, CHUNK)],
                gsem,
            )
            descs.append(d)
        for d in descs:
            d.wait()
        out_row = p0 + p * ROWS_PER_PHASE
        pltpu.sync_copy(rows_v, out_hbm.at[pl.ds(out_row, ROWS_PER_PHASE)])
        return carry

    lax.fori_loop(0, N_PHASES, phase, 0)


def _gather_rows(ids_flat, t_lin):
    mesh = plsc.VectorSubcoreMesh(
        core_axis_name="c", subcore_axis_name="s",
        num_cores=NUM_CORES, num_subcores=NUM_SUBCORES,
    )
    k = pl.kernel(
        _gather_body,
        out_type=jax.ShapeDtypeStruct((TOTAL_IDS, HIDDEN), jnp.float32),
        mesh=mesh,
        scratch_types=[
            pltpu.VMEM((RAW_N,), jnp.int32),
            pltpu.VMEM((IDS_PER_W,), jnp.int32),
            pltpu.VMEM((ROWS_PER_PHASE, HIDDEN), jnp.float32),
            pltpu.SemaphoreType.DMA,
        ],
        compiler_params=pltpu.CompilerParams(use_tc_tiling_on_sc=False,
                                             needs_layout_passes=False),
    )
    return k(ids_flat, t_lin)


LC = 2


def _delin_body(g_ref, o_ref):
    ii = lax.broadcasted_iota(jnp.int32, (HIDDEN, HIDDEN), 0)
    jj = lax.broadcasted_iota(jnp.int32, (HIDDEN, HIDDEN), 1)
    eye = jnp.where(ii == jj, 1.0, 0.0).astype(jnp.float32)
    for j in range(LC):
        g = g_ref[pl.ds(j * HB, HB), :]
        ze = lax.dot_general(eye, g[:, :HIDDEN], (((1,), (1,)), ((), ())),
                             preferred_element_type=jnp.float32)
        zo = lax.dot_general(eye, g[:, HIDDEN:], (((1,), (1,)), ((), ())),
                             preferred_element_type=jnp.float32)
        o_ref[j] = jnp.concatenate([ze, zo], axis=1)


def _delinearize(g2):
    return pl.pallas_call(
        _delin_body,
        grid=(L // LC,),
        in_specs=[pl.BlockSpec((LC * HB, 2 * HIDDEN), lambda i: (i, 0))],
        out_specs=pl.BlockSpec((LC, HIDDEN, B), lambda i: (i, 0, 0)),
        out_shape=jax.ShapeDtypeStruct((L, HIDDEN, B), jnp.float32),
    )(g2)


def kernel(input_ids, attention_mask, embed_table, W1, b1, W2, b2):
    del attention_mask
    table_t = _transform_table(embed_table.T, W1, b1, W2, b2)
    t_lin = table_t.reshape(VPAD, HIDDEN)

    ids_flat = input_ids.astype(jnp.int32).T.reshape(TOTAL_IDS)

    g = _gather_rows(ids_flat, t_lin)
    g2 = g.reshape(TOTAL_IDS // 2, 2 * HIDDEN)
    p = _delinearize(g2)
    return jnp.transpose(p, (2, 0, 1))


# speedup vs baseline: 8.7437x; 1.0546x over previous
import jax
import jax.numpy as jnp
from jax import lax
from jax.experimental import pallas as pl
from jax.experimental.pallas import tpu as pltpu
from jax.experimental.pallas import tpu_sc as plsc

VOCAB = 100000
HIDDEN = 64
B = 4096
L = 50
HB = B // 2

BLK_V = 16384
HBLK = BLK_V // 2
VPAD = 114688


def _mlp_body(e_ref, w1_ref, b1_ref, w2_ref, b2_ref, o_ref):
    e = e_ref[...]
    h = lax.dot_general(e, w1_ref[...], (((0,), (1,)), ((), ())),
                        preferred_element_type=jnp.float32)
    h = jnp.tanh(h + b1_ref[...])
    z = lax.dot_general(h, w2_ref[...], (((1,), (1,)), ((), ())),
                        preferred_element_type=jnp.float32)
    z = z + b2_ref[...]
    o_ref[...] = jnp.concatenate([z[:HBLK], z[HBLK:]], axis=1)


def _transform_table(embed_table_t, w1, b1, w2, b2):
    grid = VPAD // BLK_V
    return pl.pallas_call(
        _mlp_body,
        grid=(grid,),
        in_specs=[
            pl.BlockSpec((HIDDEN, BLK_V), lambda i: (0, i)),
            pl.BlockSpec((HIDDEN, HIDDEN), lambda i: (0, 0)),
            pl.BlockSpec((1, HIDDEN), lambda i: (0, 0)),
            pl.BlockSpec((HIDDEN, HIDDEN), lambda i: (0, 0)),
            pl.BlockSpec((1, HIDDEN), lambda i: (0, 0)),
        ],
        out_specs=pl.BlockSpec((HBLK, 2 * HIDDEN), lambda i: (i, 0)),
        out_shape=jax.ShapeDtypeStruct((VPAD // 2, 2 * HIDDEN), jnp.float32),
    )(embed_table_t, w1, b1.reshape(1, HIDDEN), w2, b2.reshape(1, HIDDEN))


NUM_CORES = 2
NUM_SUBCORES = 16
NUM_WORKERS = NUM_CORES * NUM_SUBCORES

TOTAL_IDS = B * L
CHUNK = 128
IDS_PER_W = TOTAL_IDS // NUM_WORKERS
CHUNKS_PER_W = IDS_PER_W // CHUNK
PHASE = 10
N_PHASES = CHUNKS_PER_W // PHASE
ROWS_PER_PHASE = PHASE * CHUNK
RAW_N = 3 * B
N_GROUPS = IDS_PER_W // 16


def _gather_body(ids_hbm, t_hbm, out_hbm, raw_v, idx_v, rows_v, gsem):
    wid = lax.axis_index("s") * NUM_CORES + lax.axis_index("c")
    p0 = wid * IDS_PER_W
    lbase = jnp.minimum(p0 // B, L - 3)
    pltpu.sync_copy(ids_hbm.at[pl.ds(lbase * B, RAW_N)], raw_v)

    i16 = lax.iota(jnp.int32, 16)
    pattern = HB * (i16 & 1) + (i16 >> 1)

    def build(g, carry):
        qg = p0 + 16 * g
        lrel = (qg >> 12) - lbase
        qr = qg & (B - 1)
        c0 = lrel * B + (qr >> 1)
        w = plsc.load_gather(raw_v, [pattern + c0])
        r = w & (BLK_V - 1)
        sig = (w - r) + 2 * (r & (HBLK - 1)) + (r >> 13)
        idx_v[pl.ds(16 * g, 16)] = sig
        return carry

    lax.fori_loop(0, N_GROUPS, build, 0)

    def phase(p, carry):
        descs = []
        for bb in range(PHASE):
            d = pltpu.async_copy(
                t_hbm.at[idx_v.at[pl.ds((p * PHASE + bb) * CHUNK, CHUNK)]],
                rows_v.at[pl.ds(bb * CHUNK, CHUNK)],
                gsem,
            )
            descs.append(d)
        for d in descs:
            d.wait()
        out_row = p0 + p * ROWS_PER_PHASE
        pltpu.sync_copy(rows_v, out_hbm.at[pl.ds(out_row, ROWS_PER_PHASE)])
        return carry

    lax.fori_loop(0, N_PHASES, phase, 0)


def _gather_rows(ids_flat, t_lin):
    mesh = plsc.VectorSubcoreMesh(
        core_axis_name="c", subcore_axis_name="s",
        num_cores=NUM_CORES, num_subcores=NUM_SUBCORES,
    )
    k = pl.kernel(
        _gather_body,
        out_type=jax.ShapeDtypeStruct((TOTAL_IDS, HIDDEN), jnp.float32),
        mesh=mesh,
        scratch_types=[
            pltpu.VMEM((RAW_N,), jnp.int32),
            pltpu.VMEM((IDS_PER_W,), jnp.int32),
            pltpu.VMEM((ROWS_PER_PHASE, HIDDEN), jnp.float32),
            pltpu.SemaphoreType.DMA,
        ],
        compiler_params=pltpu.CompilerParams(use_tc_tiling_on_sc=False,
                                             needs_layout_passes=False),
    )
    return k(ids_flat, t_lin)


LC = 5


def _delin_body(g_ref, o_ref):
    ii = lax.broadcasted_iota(jnp.int32, (HIDDEN, HIDDEN), 0)
    jj = lax.broadcasted_iota(jnp.int32, (HIDDEN, HIDDEN), 1)
    eye = jnp.where(ii == jj, 1.0, 0.0).astype(jnp.float32)
    for j in range(LC):
        g = g_ref[pl.ds(j * HB, HB), :]
        ze = lax.dot_general(eye, g[:, :HIDDEN], (((1,), (1,)), ((), ())),
                             preferred_element_type=jnp.float32)
        zo = lax.dot_general(eye, g[:, HIDDEN:], (((1,), (1,)), ((), ())),
                             preferred_element_type=jnp.float32)
        o_ref[j] = jnp.concatenate([ze, zo], axis=1)


def _delinearize(g2):
    return pl.pallas_call(
        _delin_body,
        grid=(L // LC,),
        in_specs=[pl.BlockSpec((LC * HB, 2 * HIDDEN), lambda i: (i, 0))],
        out_specs=pl.BlockSpec((LC, HIDDEN, B), lambda i: (i, 0, 0)),
        out_shape=jax.ShapeDtypeStruct((L, HIDDEN, B), jnp.float32),
    )(g2)


def kernel(input_ids, attention_mask, embed_table, W1, b1, W2, b2):
    del attention_mask
    table_t = _transform_table(embed_table.T, W1, b1, W2, b2)
    t_lin = table_t.reshape(VPAD, HIDDEN)

    ids_flat = input_ids.astype(jnp.int32).T.reshape(TOTAL_IDS)

    g = _gather_rows(ids_flat, t_lin)
    g2 = g.reshape(TOTAL_IDS // 2, 2 * HIDDEN)
    p = _delinearize(g2)
    return jnp.transpose(p, (2, 0, 1))


# speedup vs baseline: 8.7862x; 1.0049x over previous
import jax
import jax.numpy as jnp
from jax import lax
from jax.experimental import pallas as pl
from jax.experimental.pallas import tpu as pltpu
from jax.experimental.pallas import tpu_sc as plsc

VOCAB = 100000
HIDDEN = 64
B = 4096
L = 50
HB = B // 2

BLK_V = 16384
HBLK = BLK_V // 2
VPAD = 114688


def _mlp_body(e_ref, w1_ref, b1_ref, w2_ref, b2_ref, o_ref):
    e = e_ref[...]
    h = lax.dot_general(e, w1_ref[...], (((0,), (1,)), ((), ())),
                        preferred_element_type=jnp.float32)
    h = jnp.tanh(h + b1_ref[...])
    z = lax.dot_general(h, w2_ref[...], (((1,), (1,)), ((), ())),
                        preferred_element_type=jnp.float32)
    z = z + b2_ref[...]
    o_ref[...] = jnp.concatenate([z[:HBLK], z[HBLK:]], axis=1)


def _transform_table(embed_table_t, w1, b1, w2, b2):
    grid = VPAD // BLK_V
    return pl.pallas_call(
        _mlp_body,
        grid=(grid,),
        in_specs=[
            pl.BlockSpec((HIDDEN, BLK_V), lambda i: (0, i)),
            pl.BlockSpec((HIDDEN, HIDDEN), lambda i: (0, 0)),
            pl.BlockSpec((1, HIDDEN), lambda i: (0, 0)),
            pl.BlockSpec((HIDDEN, HIDDEN), lambda i: (0, 0)),
            pl.BlockSpec((1, HIDDEN), lambda i: (0, 0)),
        ],
        out_specs=pl.BlockSpec((HBLK, 2 * HIDDEN), lambda i: (i, 0)),
        out_shape=jax.ShapeDtypeStruct((VPAD // 2, 2 * HIDDEN), jnp.float32),
    )(embed_table_t, w1, b1.reshape(1, HIDDEN), w2, b2.reshape(1, HIDDEN))


NUM_CORES = 2
NUM_SUBCORES = 16
NUM_WORKERS = NUM_CORES * NUM_SUBCORES

TOTAL_IDS = B * L
CHUNK = 128
IDS_PER_W = TOTAL_IDS // NUM_WORKERS
CHUNKS_PER_W = IDS_PER_W // CHUNK
PHASE = 10
N_PHASES = CHUNKS_PER_W // PHASE
ROWS_PER_PHASE = PHASE * CHUNK
RAW_N = 3 * B
N_GROUPS = IDS_PER_W // 16


def _gather_body(ids_hbm, t_hbm, out_hbm, raw_v, idx_v, rows_v, gsem):
    wid = lax.axis_index("s") * NUM_CORES + lax.axis_index("c")
    p0 = wid * IDS_PER_W
    lbase = jnp.minimum(p0 // B, L - 3)
    pltpu.sync_copy(ids_hbm.at[pl.ds(lbase * B, RAW_N)], raw_v)

    i16 = lax.iota(jnp.int32, 16)
    pattern = HB * (i16 & 1) + (i16 >> 1)

    def build(g, carry):
        qg = p0 + 16 * g
        lrel = (qg >> 12) - lbase
        qr = qg & (B - 1)
        c0 = lrel * B + (qr >> 1)
        w = plsc.load_gather(raw_v, [pattern + c0])
        r = w & (BLK_V - 1)
        sig = (w - r) + 2 * (r & (HBLK - 1)) + (r >> 13)
        idx_v[pl.ds(16 * g, 16)] = sig
        return carry

    lax.fori_loop(0, N_GROUPS, build, 0)

    def phase(p, carry):
        descs = []
        for bb in range(PHASE):
            d = pltpu.async_copy(
                t_hbm.at[idx_v.at[pl.ds((p * PHASE + bb) * CHUNK, CHUNK)]],
                rows_v.at[pl.ds(bb * CHUNK, CHUNK)],
                gsem,
            )
            descs.append(d)
        for d in descs:
            d.wait()
        out_row = p0 + p * ROWS_PER_PHASE
        pltpu.sync_copy(rows_v, out_hbm.at[pl.ds(out_row, ROWS_PER_PHASE)])
        return carry

    lax.fori_loop(0, N_PHASES, phase, 0)


def _gather_rows(ids_flat, t_lin):
    mesh = plsc.VectorSubcoreMesh(
        core_axis_name="c", subcore_axis_name="s",
        num_cores=NUM_CORES, num_subcores=NUM_SUBCORES,
    )
    k = pl.kernel(
        _gather_body,
        out_type=jax.ShapeDtypeStruct((TOTAL_IDS, HIDDEN), jnp.float32),
        mesh=mesh,
        scratch_types=[
            pltpu.VMEM((RAW_N,), jnp.int32),
            pltpu.VMEM((IDS_PER_W,), jnp.int32),
            pltpu.VMEM((ROWS_PER_PHASE, HIDDEN), jnp.float32),
            pltpu.SemaphoreType.DMA,
        ],
        compiler_params=pltpu.CompilerParams(use_tc_tiling_on_sc=False,
                                             needs_layout_passes=False),
    )
    return k(ids_flat, t_lin)


LC = 10


def _delin_body(g_ref, o_ref):
    ii = lax.broadcasted_iota(jnp.int32, (HIDDEN, HIDDEN), 0)
    jj = lax.broadcasted_iota(jnp.int32, (HIDDEN, HIDDEN), 1)
    eye = jnp.where(ii == jj, 1.0, 0.0).astype(jnp.float32)
    for j in range(LC):
        g = g_ref[pl.ds(j * HB, HB), :]
        ze = lax.dot_general(eye, g[:, :HIDDEN], (((1,), (1,)), ((), ())),
                             preferred_element_type=jnp.float32)
        zo = lax.dot_general(eye, g[:, HIDDEN:], (((1,), (1,)), ((), ())),
                             preferred_element_type=jnp.float32)
        o_ref[j] = jnp.concatenate([ze, zo], axis=1)


def _delinearize(g2):
    return pl.pallas_call(
        _delin_body,
        grid=(L // LC,),
        in_specs=[pl.BlockSpec((LC * HB, 2 * HIDDEN), lambda i: (i, 0))],
        out_specs=pl.BlockSpec((LC, HIDDEN, B), lambda i: (i, 0, 0)),
        out_shape=jax.ShapeDtypeStruct((L, HIDDEN, B), jnp.float32),
    )(g2)


def kernel(input_ids, attention_mask, embed_table, W1, b1, W2, b2):
    del attention_mask
    table_t = _transform_table(embed_table.T, W1, b1, W2, b2)
    t_lin = table_t.reshape(VPAD, HIDDEN)

    ids_flat = input_ids.astype(jnp.int32).T.reshape(TOTAL_IDS)

    g = _gather_rows(ids_flat, t_lin)
    g2 = g.reshape(TOTAL_IDS // 2, 2 * HIDDEN)
    p = _delinearize(g2)
    return jnp.transpose(p, (2, 0, 1))
